# baseline (speedup 1.0000x reference)
import jax
import jax.numpy as jnp
from jax.experimental import pallas as pl

_T, _N, _F, _TILE, _A, _K, _U = 16, 8, 6, 300, 21, 20, 100
_OUT = _TILE - _K + 1
_TNF = _T * _N * _F
_BM = 16


def _r_kernel(p_ref, q_ref, r_ref):
    p = p_ref[...]
    m = jnp.max(p, axis=1, keepdims=True)
    lse = jnp.log(jnp.sum(jnp.exp(p - m), axis=1, keepdims=True)) + m
    logq = jnp.log(q_ref[...].reshape(_A))
    r_ref[...] = p - lse - logq[None, :, None]


def _conv_kernel(x_ref, r_ref, o_ref):
    x = x_ref[...]
    r = r_ref[...]
    patches = jnp.concatenate(
        [x[:, j:j + _OUT, :] for j in range(_K)], axis=2)
    pm = patches.reshape(_BM * _OUT, _K * _A)
    acc = jnp.dot(pm, r, preferred_element_type=jnp.float32)
    o_ref[...] = acc.reshape(_BM, _OUT, _U)


def kernel(X, P_logit, Q):
    r3 = pl.pallas_call(
        _r_kernel,
        out_shape=jax.ShapeDtypeStruct((_K, _A, _U), jnp.float32),
    )(P_logit, Q.reshape(1, _A))
    r2 = r3.reshape(_K * _A, _U)

    xr = X.reshape(_TNF, _TILE, _A)
    z = pl.pallas_call(
        _conv_kernel,
        grid=(_TNF // _BM,),
        in_specs=[
            pl.BlockSpec((_BM, _TILE, _A), lambda i: (i, 0, 0)),
            pl.BlockSpec((_K * _A, _U), lambda i: (0, 0)),
        ],
        out_specs=pl.BlockSpec((_BM, _OUT, _U), lambda i: (i, 0, 0)),
        out_shape=jax.ShapeDtypeStruct((_TNF, _OUT, _U), jnp.float32),
    )(xr, r2)
    return z.reshape(_T, _N, _F, _OUT, _U)
